# baseline (speedup 1.0000x reference)
import functools
import math

import jax
import jax.numpy as jnp
import numpy as np
from jax import lax
from jax.experimental import pallas as pl
from jax.experimental.pallas import tpu as pltpu
from jax.experimental.pallas import tpu_sc as plsc

N_LEVELS = 16
N_FEATURES = 2
MIN_RES = 16
MAX_RES = 2048
LOG2_T = 20
T = 1 << LOG2_T
HASH_MASK = T - 1
PRIME_I32 = np.int32(2654435761 - (1 << 32))
_GROWTH = math.exp((math.log(MAX_RES) - math.log(MIN_RES)) / (N_LEVELS - 1))
RES = [int(math.floor(MIN_RES * (_GROWTH ** l))) for l in range(N_LEVELS)]

D_ENC = N_LEVELS * N_FEATURES


def _sc_encode(u, v, tab):
    B = u.shape[0]
    info = plsc.get_sparse_core_info()
    NC, NS, L = info.num_cores, info.num_subcores, info.num_lanes
    NW = NC * NS
    assert B % NW == 0
    PPW = B // NW
    C = 512
    assert PPW % C == 0
    NCHUNK = PPW // C

    mesh = plsc.VectorSubcoreMesh(core_axis_name="c", subcore_axis_name="s")

    @functools.partial(
        pl.kernel,
        out_type=jax.ShapeDtypeStruct((D_ENC, B), jnp.float32),
        mesh=mesh,
        scratch_types=[
            pltpu.VMEM((C,), jnp.float32),
            pltpu.VMEM((C,), jnp.float32),
            pltpu.VMEM((C,), jnp.float32),
            pltpu.VMEM((C,), jnp.float32),
            pltpu.VMEM((8 * C,), jnp.int32),
            pltpu.VMEM((8 * C,), jnp.float32),
            pltpu.VMEM((D_ENC, C), jnp.float32),
            pltpu.SemaphoreType.DMA,
        ],
    )
    def enc_kernel(u_hbm, v_hbm, tab_hbm, out_hbm, u_v, v_v, wx_v, wy_v,
                   idx_v, feat_v, enc_v, sem):
        wid = lax.axis_index("s") * NC + lax.axis_index("c")

        def chunk_body(ci, _):
            base = wid * PPW + ci * C
            pltpu.sync_copy(u_hbm.at[pl.ds(base, C)], u_v)
            pltpu.sync_copy(v_hbm.at[pl.ds(base, C)], v_v)
            for l in range(N_LEVELS):
                res = float(RES[l])
                lofs2 = 2 * l * T

                def idx_body(i, _):
                    off = i * L
                    uu = u_v[pl.ds(off, L)]
                    vv = v_v[pl.ds(off, L)]
                    px = uu * res
                    py = vv * res
                    ix = px.astype(jnp.int32)
                    iy = py.astype(jnp.int32)
                    wx_v[pl.ds(off, L)] = px - ix.astype(jnp.float32)
                    wy_v[pl.ds(off, L)] = py - iy.astype(jnp.float32)
                    hy0 = iy * PRIME_I32
                    hy1 = hy0 + PRIME_I32
                    ix1 = ix + 1
                    e00 = 2 * ((ix ^ hy0) & HASH_MASK) + lofs2
                    e10 = 2 * ((ix1 ^ hy0) & HASH_MASK) + lofs2
                    e01 = 2 * ((ix ^ hy1) & HASH_MASK) + lofs2
                    e11 = 2 * ((ix1 ^ hy1) & HASH_MASK) + lofs2
                    idx_v[pl.ds(0 * C + off, L)] = e00
                    idx_v[pl.ds(1 * C + off, L)] = e00 + 1
                    idx_v[pl.ds(2 * C + off, L)] = e10
                    idx_v[pl.ds(3 * C + off, L)] = e10 + 1
                    idx_v[pl.ds(4 * C + off, L)] = e01
                    idx_v[pl.ds(5 * C + off, L)] = e01 + 1
                    idx_v[pl.ds(6 * C + off, L)] = e11
                    idx_v[pl.ds(7 * C + off, L)] = e11 + 1
                    return 0

                lax.fori_loop(0, C // L, idx_body, 0)
                pltpu.async_copy(tab_hbm.at[idx_v], feat_v, sem).wait()

                def blend_body(i, _):
                    off = i * L
                    wx = wx_v[pl.ds(off, L)]
                    wy = wy_v[pl.ds(off, L)]
                    for f in range(N_FEATURES):
                        f00 = feat_v[pl.ds((0 + f) * C + off, L)]
                        f10 = feat_v[pl.ds((2 + f) * C + off, L)]
                        f01 = feat_v[pl.ds((4 + f) * C + off, L)]
                        f11 = feat_v[pl.ds((6 + f) * C + off, L)]
                        a = f00 + wx * (f10 - f00)
                        b = f01 + wx * (f11 - f01)
                        enc_v[N_FEATURES * l + f, pl.ds(off, L)] = (
                            a + wy * (b - a))
                    return 0

                lax.fori_loop(0, C // L, blend_body, 0)
            pltpu.sync_copy(enc_v, out_hbm.at[:, pl.ds(base, C)])
            return 0

        lax.fori_loop(0, NCHUNK, chunk_body, 0)

    return enc_kernel(u, v, tab)


def _mlp_call(enc_t, W0T, W1T, W2T):
    B = enc_t.shape[1]
    BT = 4096
    assert B % BT == 0

    def mlp_kernel(e_ref, w0_ref, w1_ref, w2_ref, o_ref):
        h = jnp.dot(w0_ref[...], e_ref[...], preferred_element_type=jnp.float32)
        h = jnp.maximum(h, 0.0)
        h = jnp.dot(w1_ref[...], h, preferred_element_type=jnp.float32)
        h = jnp.maximum(h, 0.0)
        o = jnp.dot(w2_ref[...], h, preferred_element_type=jnp.float32)
        o_ref[...] = jax.nn.sigmoid(o)

    return pl.pallas_call(
        mlp_kernel,
        grid=(B // BT,),
        in_specs=[
            pl.BlockSpec((D_ENC, BT), lambda i: (0, i)),
            pl.BlockSpec((64, D_ENC), lambda i: (0, 0)),
            pl.BlockSpec((64, 64), lambda i: (0, 0)),
            pl.BlockSpec((3, 64), lambda i: (0, 0)),
        ],
        out_specs=pl.BlockSpec((3, BT), lambda i: (0, i)),
        out_shape=jax.ShapeDtypeStruct((3, B), jnp.float32),
    )(enc_t, W0T, W1T, W2T)


def kernel(uv, tables, W0, W1, W2):
    u = uv[:, 0]
    v = uv[:, 1]
    tab = tables.reshape(N_LEVELS * T * N_FEATURES)
    enc_t = _sc_encode(u, v, tab)
    out_t = _mlp_call(enc_t, W0.T, W1.T, W2.T)
    return out_t.T
